# Initial kernel scaffold; baseline (speedup 1.0000x reference)
import jax
import jax.numpy as jnp
from jax.experimental import pallas as pl


def kernel(x, edge_index, W, a1_w, a1_b, a2_w, a2_b):
    raise NotImplementedError("write your pallas kernel here")


# speedup vs baseline: 75.7522x; 75.7522x over previous
import functools

import jax
import jax.numpy as jnp
from jax import lax
from jax.experimental import pallas as pl
from jax.experimental.pallas import tpu as pltpu
from jax.experimental.pallas import tpu_sc as plsc

N = 10000
NP = 10240
E = 320000
D = 128
H = 8
HD = 16
NC = 2
NS = 16
HC = H // NC
DH = D // NC
L = 16
NPT = NP // NS
EPT = E // NS
B = 80
NWIN = EPT // B

_f32 = jnp.float32


def _prep_body(x_ref, wall_ref, a1_ref, b1_ref, a2_ref, b2_ref,
               seq2_ref, f1_ref, f2_ref):
    s = jnp.dot(x_ref[...], wall_ref[...], preferred_element_type=_f32)
    f1 = jnp.dot(s, a1_ref[...], preferred_element_type=_f32) + b1_ref[...]
    f2 = jnp.dot(s, a2_ref[...], preferred_element_type=_f32) + b2_ref[...]
    seq2_ref[0] = s[:, :DH]
    seq2_ref[1] = s[:, DH:]
    f1_ref[0] = jnp.concatenate([f1[:, :HC]] * (L // HC), axis=1)
    f1_ref[1] = jnp.concatenate([f1[:, HC:]] * (L // HC), axis=1)
    f2_ref[0] = jnp.concatenate([f2[:, :HC]] * (L // HC), axis=1)
    f2_ref[1] = jnp.concatenate([f2[:, HC:]] * (L // HC), axis=1)


def _prep(x, wall, a1, b1, a2, b2):
    grid = 10
    rb = NP // grid
    return pl.pallas_call(
        _prep_body,
        grid=(grid,),
        in_specs=[
            pl.BlockSpec((rb, D), lambda i: (i, 0)),
            pl.BlockSpec((D, D), lambda i: (0, 0)),
            pl.BlockSpec((D, H), lambda i: (0, 0)),
            pl.BlockSpec((1, H), lambda i: (0, 0)),
            pl.BlockSpec((D, H), lambda i: (0, 0)),
            pl.BlockSpec((1, H), lambda i: (0, 0)),
        ],
        out_specs=[
            pl.BlockSpec((NC, rb, DH), lambda i: (0, i, 0)),
            pl.BlockSpec((NC, rb, L), lambda i: (0, i, 0)),
            pl.BlockSpec((NC, rb, L), lambda i: (0, i, 0)),
        ],
        out_shape=[
            jax.ShapeDtypeStruct((NC, NP, DH), _f32),
            jax.ShapeDtypeStruct((NC, NP, L), _f32),
            jax.ShapeDtypeStruct((NC, NP, L), _f32),
        ],
    )(x, wall, a1, b1, a2, b2)


def _sc_body(seq2_hbm, f1_hbm, f2_hbm, row_hbm, col_hbm, z64_hbm, z16_hbm,
             num_out, den_out,
             num_s, sf_s, f1_s, f2_s, den_s,
             rowv, colv, g1v, g2v, sfv, exv):
    c = lax.axis_index("c")
    s = lax.axis_index("s")
    r0 = s * NPT

    pltpu.sync_copy(z64_hbm.at[pl.ds(r0, NPT)], num_s.at[pl.ds(r0, NPT)])
    pltpu.sync_copy(z16_hbm.at[pl.ds(r0, NPT)], den_s.at[pl.ds(r0, NPT)])
    pltpu.sync_copy(seq2_hbm.at[c, pl.ds(r0, NPT)], sf_s.at[pl.ds(r0, NPT)])
    pltpu.sync_copy(f1_hbm.at[c, pl.ds(r0, NPT)], f1_s.at[pl.ds(r0, NPT)])
    pltpu.sync_copy(f2_hbm.at[c, pl.ds(r0, NPT)], f2_s.at[pl.ds(r0, NPT)])
    plsc.subcore_barrier()

    t0 = s * EPT

    def window(wi, carry):
        base = t0 + wi * B
        pltpu.sync_copy(row_hbm.at[pl.ds(base, B)], rowv)
        pltpu.sync_copy(col_hbm.at[pl.ds(base, B)], colv)
        pltpu.sync_copy(f1_s.at[rowv], g1v)
        pltpu.sync_copy(f2_s.at[colv], g2v)
        pltpu.sync_copy(sf_s.at[colv], sfv)

        def edge(e, carry2):
            w = g1v[e] + g2v[e]
            w = jnp.maximum(w * jnp.float32(0.2), w)
            ex = jnp.exp(w)
            exv[e] = ex
            for j in range(HC):
                sfv[e, pl.ds(HD * j, HD)] = sfv[e, pl.ds(HD * j, HD)] * ex[j]
            return carry2

        lax.fori_loop(0, B, edge, 0)
        pltpu.sync_copy(sfv, num_s.at[rowv], add=True)
        pltpu.sync_copy(exv, den_s.at[rowv], add=True)
        return carry

    lax.fori_loop(0, NWIN, window, 0)
    plsc.subcore_barrier()

    pltpu.sync_copy(num_s.at[pl.ds(r0, NPT)], num_out.at[c, pl.ds(r0, NPT)])
    pltpu.sync_copy(den_s.at[pl.ds(r0, NPT)], den_out.at[c, pl.ds(r0, NPT)])


_sc_call = functools.partial(
    pl.kernel,
    out_type=(
        jax.ShapeDtypeStruct((NC, NP, DH), _f32),
        jax.ShapeDtypeStruct((NC, NP, L), _f32),
    ),
    mesh=plsc.VectorSubcoreMesh(core_axis_name="c", subcore_axis_name="s"),
    compiler_params=pltpu.CompilerParams(use_tc_tiling_on_sc=False),
    scratch_types=[
        pltpu.VMEM_SHARED((NP, DH), _f32),
        pltpu.VMEM_SHARED((NP, DH), _f32),
        pltpu.VMEM_SHARED((NP, L), _f32),
        pltpu.VMEM_SHARED((NP, L), _f32),
        pltpu.VMEM_SHARED((NP, L), _f32),
        pltpu.VMEM((B,), jnp.int32),
        pltpu.VMEM((B,), jnp.int32),
        pltpu.VMEM((B, L), _f32),
        pltpu.VMEM((B, L), _f32),
        pltpu.VMEM((B, DH), _f32),
        pltpu.VMEM((B, L), _f32),
    ],
)(_sc_body)


def _fin_body(num_ref, den_ref, e0_ref, e1_ref, o_ref):
    n = jnp.concatenate([num_ref[0], num_ref[1]], axis=1)
    dex = (jnp.dot(den_ref[0], e0_ref[...], preferred_element_type=_f32)
           + jnp.dot(den_ref[1], e1_ref[...], preferred_element_type=_f32))
    h = jnp.where(dex > 0, n / jnp.where(dex > 0, dex, 1.0), 0.0)
    o_ref[...] = jnp.where(h > 0, h, jnp.exp(h) - 1.0)


def _finish(num, den, e0, e1):
    grid = 10
    rb = NP // grid
    return pl.pallas_call(
        _fin_body,
        grid=(grid,),
        in_specs=[
            pl.BlockSpec((NC, rb, DH), lambda i: (0, i, 0)),
            pl.BlockSpec((NC, rb, L), lambda i: (0, i, 0)),
            pl.BlockSpec((L, D), lambda i: (0, 0)),
            pl.BlockSpec((L, D), lambda i: (0, 0)),
        ],
        out_specs=pl.BlockSpec((rb, D), lambda i: (i, 0)),
        out_shape=jax.ShapeDtypeStruct((NP, D), _f32),
    )(num, den, e0, e1)


def kernel(x, edge_index, W, a1_w, a1_b, a2_w, a2_b):
    row = edge_index[0].astype(jnp.int32)
    col = edge_index[1].astype(jnp.int32)

    wall = jnp.transpose(W, (1, 0, 2)).reshape(D, D)
    eye = jnp.eye(H, dtype=_f32)
    a1 = (eye[:, None, :] * a1_w[:, :, 0][:, :, None]).reshape(D, H)
    a2 = (eye[:, None, :] * a2_w[:, :, 0][:, :, None]).reshape(D, H)
    b1 = a1_b[:, 0][None, :]
    b2 = a2_b[:, 0][None, :]

    xp = jnp.pad(x, ((0, NP - N), (0, 0)))
    seq2, f1t, f2t = _prep(xp, wall, a1, b1, a2, b2)

    z64 = jnp.zeros((NP, DH), _f32)
    z16 = jnp.zeros((NP, L), _f32)
    num, den = _sc_call(seq2, f1t, f2t, row, col, z64, z16)

    lane = jnp.arange(L, dtype=jnp.int32)[:, None]
    headcol = (jnp.arange(D, dtype=jnp.int32) // HD)[None, :]
    e0 = ((lane == headcol) & (lane < HC)).astype(_f32)
    e1 = ((lane == (headcol - HC)) & (lane < HC)).astype(_f32)

    return _finish(num, den, e0, e1)[:N]


# speedup vs baseline: 130.5453x; 1.7233x over previous
import functools

import jax
import jax.numpy as jnp
from jax import lax
from jax.experimental import pallas as pl
from jax.experimental.pallas import tpu as pltpu
from jax.experimental.pallas import tpu_sc as plsc

N = 10000
NP = 10240
E = 320000
D = 128
H = 8
HD = 16
NC = 2
NS = 16
HC = H // NC
DH = D // NC
L = 16
SW = DH + L
NPT = NP // NS
EPT = E // NS
B = 80
NWIN = EPT // B

_f32 = jnp.float32


def _prep_body(x_ref, wall_ref, a1_ref, b1_ref, a2_ref, b2_ref,
               sft_ref, f1_ref):
    s = jnp.dot(x_ref[...], wall_ref[...], preferred_element_type=_f32)
    f1 = jnp.dot(s, a1_ref[...], preferred_element_type=_f32) + b1_ref[...]
    f2 = jnp.dot(s, a2_ref[...], preferred_element_type=_f32) + b2_ref[...]
    rep = L // HC
    sft_ref[0] = jnp.concatenate([s[:, :DH]] + [f2[:, :HC]] * rep, axis=1)
    sft_ref[1] = jnp.concatenate([s[:, DH:]] + [f2[:, HC:]] * rep, axis=1)
    f1_ref[0] = jnp.concatenate([f1[:, :HC]] * rep, axis=1)
    f1_ref[1] = jnp.concatenate([f1[:, HC:]] * rep, axis=1)


def _prep(x, wall, a1, b1, a2, b2):
    grid = 10
    rb = NP // grid
    return pl.pallas_call(
        _prep_body,
        grid=(grid,),
        in_specs=[
            pl.BlockSpec((rb, D), lambda i: (i, 0)),
            pl.BlockSpec((D, D), lambda i: (0, 0)),
            pl.BlockSpec((D, H), lambda i: (0, 0)),
            pl.BlockSpec((1, H), lambda i: (0, 0)),
            pl.BlockSpec((D, H), lambda i: (0, 0)),
            pl.BlockSpec((1, H), lambda i: (0, 0)),
        ],
        out_specs=[
            pl.BlockSpec((NC, rb, SW), lambda i: (0, i, 0)),
            pl.BlockSpec((NC, rb, L), lambda i: (0, i, 0)),
        ],
        out_shape=[
            jax.ShapeDtypeStruct((NC, NP, SW), _f32),
            jax.ShapeDtypeStruct((NC, NP, L), _f32),
        ],
    )(x, wall, a1, b1, a2, b2)


def _sc_body(sft_hbm, f1_hbm, row_hbm, col_hbm, z80_hbm,
             acc_out,
             acc_s, sft_s, f1_s,
             rowv, colv, srow, g1v, sfv, isem, gsem, ssem):
    c = lax.axis_index("c")
    s = lax.axis_index("s")
    r0 = s * NPT

    pltpu.sync_copy(z80_hbm.at[pl.ds(r0, NPT)], acc_s.at[pl.ds(r0, NPT)])
    pltpu.sync_copy(sft_hbm.at[c, pl.ds(r0, NPT)], sft_s.at[pl.ds(r0, NPT)])
    pltpu.sync_copy(f1_hbm.at[c, pl.ds(r0, NPT)], f1_s.at[pl.ds(r0, NPT)])
    plsc.subcore_barrier()

    def start_idx(w, b):
        pltpu.async_copy(row_hbm.at[s, w], rowv.at[b], isem.at[b])
        pltpu.async_copy(col_hbm.at[s, w], colv.at[b], isem.at[b])

    def wait_idx(w, b):
        pltpu.make_async_copy(row_hbm.at[s, w], rowv.at[b], isem.at[b]).wait()
        pltpu.make_async_copy(col_hbm.at[s, w], colv.at[b], isem.at[b]).wait()

    def start_gathers(b):
        pltpu.async_copy(f1_s.at[rowv.at[b]], g1v.at[b], gsem.at[b])
        pltpu.async_copy(sft_s.at[colv.at[b]], sfv.at[b], gsem.at[b])

    def wait_gathers(b):
        pltpu.make_async_copy(f1_s.at[rowv.at[b]], g1v.at[b], gsem.at[b]).wait()
        pltpu.make_async_copy(sft_s.at[colv.at[b]], sfv.at[b], gsem.at[b]).wait()

    def start_scatter(b):
        pltpu.async_copy(sfv.at[b], acc_s.at[srow.at[b]], ssem.at[b], add=True)

    def wait_scatter(b):
        pltpu.make_async_copy(sfv.at[b], acc_s.at[srow.at[b]], ssem.at[b]).wait()

    def compute(b):
        g1b, sfb = g1v.at[b], sfv.at[b]

        def edge(e, carry2):
            w = g1b[e] + sfb[e, pl.ds(DH, L)]
            w = jnp.maximum(w * jnp.float32(0.2), w)
            ex = jnp.exp(w)
            for j in range(HC):
                sfb[e, pl.ds(HD * j, HD)] = sfb[e, pl.ds(HD * j, HD)] * ex[j]
            sfb[e, pl.ds(DH, L)] = ex
            return carry2

        lax.fori_loop(0, B, edge, 0, unroll=2)

    start_idx(0, 0)
    wait_idx(0, 0)
    start_gathers(0)

    def body(wo, carry):
        for b in range(2):
            w = 2 * wo + b
            wait_gathers(b)
            for k in range(B // L):
                srow[b, pl.ds(k * L, L)] = rowv[b, pl.ds(k * L, L)]

            @pl.when(w + 1 < NWIN)
            def _():
                start_idx(w + 1, 1 - b)

            compute(b)
            start_scatter(b)

            @pl.when(w >= 1)
            def _():
                wait_scatter(1 - b)

            @pl.when(w + 1 < NWIN)
            def _():
                wait_idx(w + 1, 1 - b)
                start_gathers(1 - b)

        return carry

    lax.fori_loop(0, NWIN // 2, body, 0)
    wait_scatter((NWIN - 1) % 2)
    plsc.subcore_barrier()

    pltpu.sync_copy(acc_s.at[pl.ds(r0, NPT)], acc_out.at[c, pl.ds(r0, NPT)])


_sc_call = functools.partial(
    pl.kernel,
    out_type=jax.ShapeDtypeStruct((NC, NP, SW), _f32),
    mesh=plsc.VectorSubcoreMesh(core_axis_name="c", subcore_axis_name="s"),
    compiler_params=pltpu.CompilerParams(use_tc_tiling_on_sc=False),
    scratch_types=[
        pltpu.VMEM_SHARED((NP, SW), _f32),
        pltpu.VMEM_SHARED((NP, SW), _f32),
        pltpu.VMEM_SHARED((NP, L), _f32),
        pltpu.VMEM((2, B), jnp.int32),
        pltpu.VMEM((2, B), jnp.int32),
        pltpu.VMEM((2, B), jnp.int32),
        pltpu.VMEM((2, B, L), _f32),
        pltpu.VMEM((2, B, SW), _f32),
        pltpu.SemaphoreType.DMA((2,)),
        pltpu.SemaphoreType.DMA((2,)),
        pltpu.SemaphoreType.DMA((2,)),
    ],
)(_sc_body)


def _fin_body(acc_ref, e0_ref, e1_ref, o_ref):
    n = jnp.concatenate([acc_ref[0, :, :DH], acc_ref[1, :, :DH]], axis=1)
    dex = (jnp.dot(acc_ref[0, :, DH:], e0_ref[...], preferred_element_type=_f32)
           + jnp.dot(acc_ref[1, :, DH:], e1_ref[...], preferred_element_type=_f32))
    h = jnp.where(dex > 0, n / jnp.where(dex > 0, dex, 1.0), 0.0)
    o_ref[...] = jnp.where(h > 0, h, jnp.exp(h) - 1.0)


def _finish(acc, e0, e1):
    grid = 10
    rb = NP // grid
    return pl.pallas_call(
        _fin_body,
        grid=(grid,),
        in_specs=[
            pl.BlockSpec((NC, rb, SW), lambda i: (0, i, 0)),
            pl.BlockSpec((L, D), lambda i: (0, 0)),
            pl.BlockSpec((L, D), lambda i: (0, 0)),
        ],
        out_specs=pl.BlockSpec((rb, D), lambda i: (i, 0)),
        out_shape=jax.ShapeDtypeStruct((NP, D), _f32),
    )(acc, e0, e1)


def kernel(x, edge_index, W, a1_w, a1_b, a2_w, a2_b):
    row = edge_index[0].astype(jnp.int32)
    col = edge_index[1].astype(jnp.int32)

    wall = jnp.transpose(W, (1, 0, 2)).reshape(D, D)
    eye = jnp.eye(H, dtype=_f32)
    a1 = (eye[:, None, :] * a1_w[:, :, 0][:, :, None]).reshape(D, H)
    a2 = (eye[:, None, :] * a2_w[:, :, 0][:, :, None]).reshape(D, H)
    b1 = a1_b[:, 0][None, :]
    b2 = a2_b[:, 0][None, :]

    xp = jnp.pad(x, ((0, NP - N), (0, 0)))
    sft, f1t = _prep(xp, wall, a1, b1, a2, b2)

    z80 = jnp.zeros((NP, SW), _f32)
    row3 = row.reshape(NS, NWIN, B)
    col3 = col.reshape(NS, NWIN, B)
    acc = _sc_call(sft, f1t, row3, col3, z80)

    lane = jnp.arange(L, dtype=jnp.int32)[:, None]
    headcol = (jnp.arange(D, dtype=jnp.int32) // HD)[None, :]
    e0 = ((lane == headcol) & (lane < HC)).astype(_f32)
    e1 = ((lane == (headcol - HC)) & (lane < HC)).astype(_f32)

    return _finish(acc, e0, e1)[:N]
